# baseline (speedup 1.0000x reference)
import functools
import math

import jax
import jax.numpy as jnp
from jax import lax
from jax.experimental import pallas as pl
from jax.experimental.pallas import tpu as pltpu
from jax.experimental.pallas import tpu_sc as plsc

N = 10000
E = 160000
D_IN = 256
D_ATTR = 8
D_EDGE = 4
N_BASIS = 10
H = 64
_SILU_NORM = 1.679177
_INV2048 = 1.0 / math.sqrt(256.0 * 8.0)
_W_SCALE = 1.0 / (8.0 * 2.0 * math.sqrt(32.0))
_C_S = math.sin(math.pi / 8.0)
_C_X = math.cos(math.pi / 8.0)

BN = 1000
BE = 640
CW = 128
K = 40
EH = E // 2
EH_PAD = 81920
ROWS_PER_TILE = 128
NCHUNK = 128
REAL_EDGE_BLOCKS = EH // BE


def _node_front_body(xin_ref, na_ref, nc_ref, wsc_ref, wl1_ref,
                     s_ref, x0_ref, x1_ref):
    xin = xin_ref[...]
    y = jnp.concatenate([na_ref[...], nc_ref[...]], axis=1)
    xcat = jnp.concatenate([xin * y[:, v:v + 1] for v in range(16)], axis=1)
    s = jnp.dot(xcat, wsc_ref[...], preferred_element_type=jnp.float32)
    x = jnp.dot(xcat, wl1_ref[...], preferred_element_type=jnp.float32)
    s_ref[...] = s * _INV2048
    x = x * _INV2048
    x0_ref[...] = x[:, :CW]
    x1_ref[...] = x[:, CW:]


def _edge_body(emb_ref, ea_ref, w0_ref, w1_ref, w2r_ref, o0_ref, o1_ref):
    emb = emb_ref[...]
    h = jnp.dot(emb, w0_ref[...], preferred_element_type=jnp.float32)
    h = h * (1.0 / math.sqrt(N_BASIS))
    h = jax.nn.silu(h) * _SILU_NORM
    h = jnp.dot(h, w1_ref[...], preferred_element_type=jnp.float32) * (1.0 / 8.0)
    h = jax.nn.silu(h) * _SILU_NORM
    ea = ea_ref[...]
    h = h * _W_SCALE
    g = jnp.concatenate([h * ea[:, v:v + 1] for v in range(D_EDGE)], axis=1)
    weff = jnp.dot(g, w2r_ref[...], preferred_element_type=jnp.float32)
    pad = pl.program_id(0) >= REAL_EDGE_BLOCKS
    weff = jnp.where(pad, 0.0, weff)
    o0_ref[...] = weff[:, :CW]
    o1_ref[...] = weff[:, CW:]


def _node_back_body(y0a_ref, y1a_ref, y0b_ref, y1b_ref, na_ref, nc_ref,
                    sym_ref, s_ref, wl2_ref, wl3_ref, out_ref):
    xm = jnp.concatenate([y0a_ref[...] + y0b_ref[...],
                          y1a_ref[...] + y1b_ref[...]], axis=1)
    a = jnp.concatenate([na_ref[...], nc_ref[...]], axis=1)
    xcat = jnp.concatenate([xm * a[:, v:v + 1] for v in range(16)], axis=1)
    x2 = jnp.dot(xcat, wl2_ref[...], preferred_element_type=jnp.float32)
    x2 = x2 * _INV2048
    sym = sym_ref[...]
    xcat2 = jnp.concatenate([x2 * sym[:, v:v + 1] for v in range(8)], axis=1)
    x3 = jnp.dot(xcat2, wl3_ref[...], preferred_element_type=jnp.float32)
    x3 = x3 * _INV2048
    out_ref[...] = _C_S * s_ref[...] + _C_X * x3


def _sc_message_body(x0_hbm, x1_hbm, w0_hbm, w1_hbm, src_hbm, dst_hbm,
                     y0_hbm, y1_hbm,
                     xg0, xg1, xg2, wv0, wv1, sb0, sb1,
                     idxs, idxd, acc,
                     gs0, gs1, gs2, ws0, ws1, ss0, ss1):
    c = lax.axis_index("c")
    s = lax.axis_index("s")
    xgs = [xg0, xg1, xg2]
    wvs = [wv0, wv1]
    sbs = [sb0, sb1]
    gss = [gs0, gs1, gs2]
    wss = [ws0, ws1]
    sss = [ss0, ss1]

    def run(xt, wt, yt):
        t_base = s * ROWS_PER_TILE

        def zb(r, carry):
            for g in range(8):
                sb0[r, pl.ds(g * 16, 16)] = jnp.zeros((16,), jnp.float32)
            return carry
        lax.fori_loop(0, K, zb, 0)
        for k in range(16):
            cid = s + 16 * k

            def zcopy(cid=cid):
                pltpu.sync_copy(sb0, acc.at[pl.ds(cid * K, K)])
            pl.when(cid < N // K)(zcopy)
        plsc.subcore_barrier()

        pltpu.sync_copy(src_hbm.at[pl.ds(t_base, 16)], idxs.at[pl.ds(0, 16)])
        pltpu.sync_copy(dst_hbm.at[pl.ds(t_base, 16)], idxd.at[pl.ds(0, 16)])
        for t in range(3):
            pltpu.async_copy(xt.at[idxs.at[t]], xgs[t], gss[t])
        for t in range(2):
            pltpu.async_copy(wt.at[pl.ds((t_base + t) * K, K)], wvs[t], wss[t])

        def load_next_group(j):
            gp = j // 16 + 1
            half = (gp % 2) * 16
            pltpu.sync_copy(src_hbm.at[pl.ds(t_base + gp * 16, 16)],
                            idxs.at[pl.ds(half, 16)])
            pltpu.sync_copy(dst_hbm.at[pl.ds(t_base + gp * 16, 16)],
                            idxd.at[pl.ds(half, 16)])

        def process(j, t3, t2, in_loop):
            if in_loop:
                pl.when((j % 16 == 8) & (j < 112))(lambda: load_next_group(j))

            def drain_sc():
                jm = (j - 2) % 32
                pltpu.make_async_copy(sbs[t2], acc.at[idxd.at[jm]],
                                      sss[t2]).wait()
            if isinstance(j, int):
                if j >= 2:
                    drain_sc()
            else:
                pl.when(j >= 2)(drain_sc)

            jr = j % 32
            pltpu.make_async_copy(xt.at[idxs.at[jr]], xgs[t3], gss[t3]).wait()
            pltpu.make_async_copy(wt.at[pl.ds((t_base + j) * K, K)],
                                  wvs[t2], wss[t2]).wait()

            def mp(r, carry):
                for g in range(8):
                    sl = pl.ds(g * 16, 16)
                    sbs[t2][r, sl] = xgs[t3][r, sl] * wvs[t2][r, sl]
                return carry
            lax.fori_loop(0, K, mp, 0)

            pltpu.async_copy(sbs[t2], acc.at[idxd.at[jr]], sss[t2], add=True)
            if in_loop:
                def issue_next_gather():
                    pltpu.async_copy(xt.at[idxs.at[(j + 3) % 32]],
                                     xgs[t3], gss[t3])
                pl.when(j + 3 < NCHUNK)(issue_next_gather)
                pltpu.async_copy(wt.at[pl.ds((t_base + j + 2) * K, K)],
                                 wvs[t2], wss[t2])
            else:
                if j + 3 < NCHUNK:
                    pltpu.async_copy(xt.at[idxs.at[(j + 3) % 32]],
                                     xgs[t3], gss[t3])
                if j + 2 < NCHUNK:
                    pltpu.async_copy(wt.at[pl.ds((t_base + j + 2) * K, K)],
                                     wvs[t2], wss[t2])

        def step(si, carry):
            for u in range(6):
                process(si * 6 + u, u % 3, u % 2, True)
            return carry
        lax.fori_loop(0, (NCHUNK // 6), step, 0)
        for j in range((NCHUNK // 6) * 6, NCHUNK):
            process(j, j % 3, j % 2, False)
        for j in range(NCHUNK - 2, NCHUNK):
            t2 = j % 2
            pltpu.make_async_copy(sbs[t2], acc.at[idxd.at[j % 64]],
                                  sss[t2]).wait()
        plsc.subcore_barrier()

        def cback(cid):
            pltpu.sync_copy(acc.at[pl.ds(cid * K, K)], sb0)
            pltpu.sync_copy(sb0, yt.at[pl.ds(cid * K, K)])
        for k in range(16):
            cid = s + 16 * k
            pl.when(cid < N // K)(lambda cid=cid: cback(cid))
        plsc.subcore_barrier()

    pl.when(c == 0)(lambda: run(x0_hbm, w0_hbm, y0_hbm))
    pl.when(c == 1)(lambda: run(x1_hbm, w1_hbm, y1_hbm))


def _tc_node_front(node_input, node_attr, node_crystal_attr, w_sc_cat, w_l1_cat):
    grid = N // BN
    return pl.pallas_call(
        _node_front_body,
        grid=(grid,),
        in_specs=[
            pl.BlockSpec((BN, D_IN), lambda i: (i, 0)),
            pl.BlockSpec((BN, D_ATTR), lambda i: (i, 0)),
            pl.BlockSpec((BN, D_ATTR), lambda i: (i, 0)),
            pl.BlockSpec((16 * D_IN, D_IN), lambda i: (0, 0)),
            pl.BlockSpec((16 * D_IN, D_IN), lambda i: (0, 0)),
        ],
        out_specs=[
            pl.BlockSpec((BN, D_IN), lambda i: (i, 0)),
            pl.BlockSpec((BN, CW), lambda i: (i, 0)),
            pl.BlockSpec((BN, CW), lambda i: (i, 0)),
        ],
        out_shape=[
            jax.ShapeDtypeStruct((N, D_IN), jnp.float32),
            jax.ShapeDtypeStruct((N, CW), jnp.float32),
            jax.ShapeDtypeStruct((N, CW), jnp.float32),
        ],
    )(node_input, node_attr, node_crystal_attr, w_sc_cat, w_l1_cat)


def _tc_edge(edge_length_embedded, edge_attr, w_fc0, w_fc1, w2r, h2):
    grid = EH_PAD // BE
    off = h2 * REAL_EDGE_BLOCKS
    return pl.pallas_call(
        _edge_body,
        grid=(grid,),
        in_specs=[
            pl.BlockSpec((BE, N_BASIS),
                         lambda i: (off + jnp.minimum(i, REAL_EDGE_BLOCKS - 1), 0)),
            pl.BlockSpec((BE, D_EDGE),
                         lambda i: (off + jnp.minimum(i, REAL_EDGE_BLOCKS - 1), 0)),
            pl.BlockSpec((N_BASIS, H), lambda i: (0, 0)),
            pl.BlockSpec((H, H), lambda i: (0, 0)),
            pl.BlockSpec((D_EDGE * H, D_IN), lambda i: (0, 0)),
        ],
        out_specs=[
            pl.BlockSpec((BE, CW), lambda i: (i, 0)),
            pl.BlockSpec((BE, CW), lambda i: (i, 0)),
        ],
        out_shape=[
            jax.ShapeDtypeStruct((EH_PAD, CW), jnp.float32),
            jax.ShapeDtypeStruct((EH_PAD, CW), jnp.float32),
        ],
    )(edge_length_embedded, edge_attr, w_fc0, w_fc1, w2r)


def _sc_message(x0, x1, w0, w1, src2, dst2):
    mesh = plsc.VectorSubcoreMesh(core_axis_name="c", subcore_axis_name="s")
    f = functools.partial(
        pl.kernel,
        out_type=[
            jax.ShapeDtypeStruct((N, CW), jnp.float32),
            jax.ShapeDtypeStruct((N, CW), jnp.float32),
        ],
        mesh=mesh,
        scratch_types=(
            [pltpu.VMEM((K, CW), jnp.float32) for _ in range(3)]
            + [pltpu.VMEM((K, CW), jnp.float32) for _ in range(2)]
            + [pltpu.VMEM((K, CW), jnp.float32) for _ in range(2)]
            + [pltpu.VMEM((32, K), jnp.int32) for _ in range(2)]
            + [pltpu.VMEM_SHARED((N, CW), jnp.float32)]
            + [pltpu.SemaphoreType.DMA for _ in range(7)]
        ),
    )(_sc_message_body)
    return f(x0, x1, w0, w1, src2, dst2)


def _tc_node_back(ys, node_attr, node_crystal_attr, sym_mask, s,
                  w_l2_cat, w_lin3):
    grid = N // BN
    return pl.pallas_call(
        _node_back_body,
        grid=(grid,),
        in_specs=[
            pl.BlockSpec((BN, CW), lambda i: (i, 0)),
            pl.BlockSpec((BN, CW), lambda i: (i, 0)),
            pl.BlockSpec((BN, CW), lambda i: (i, 0)),
            pl.BlockSpec((BN, CW), lambda i: (i, 0)),
            pl.BlockSpec((BN, D_ATTR), lambda i: (i, 0)),
            pl.BlockSpec((BN, D_ATTR), lambda i: (i, 0)),
            pl.BlockSpec((BN, D_ATTR), lambda i: (i, 0)),
            pl.BlockSpec((BN, D_IN), lambda i: (i, 0)),
            pl.BlockSpec((16 * D_IN, D_IN), lambda i: (0, 0)),
            pl.BlockSpec((D_ATTR * D_IN, D_IN), lambda i: (0, 0)),
        ],
        out_specs=pl.BlockSpec((BN, D_IN), lambda i: (i, 0)),
        out_shape=jax.ShapeDtypeStruct((N, D_IN), jnp.float32),
    )(*ys, node_attr, node_crystal_attr, sym_mask, s, w_l2_cat, w_lin3)


def kernel(node_input, node_attr, edge_attr, edge_length_embedded,
           node_crystal_attr, sym_mask, W_sc_attr, W_sc_crystal, W_lin1_attr,
           W_lin1_crystal, W_fc0, W_fc1, W_fc2, W_lin2_attr, W_lin2_crystal,
           W_lin3, edge_src, edge_dst):
    def vmajor(wa, wb):
        w = jnp.concatenate([wa, wb], axis=1)
        return w.transpose(1, 0, 2).reshape(-1, D_IN)

    w_sc_cat = vmajor(W_sc_attr, W_sc_crystal)
    w_l1_cat = vmajor(W_lin1_attr, W_lin1_crystal)
    w_l2_cat = vmajor(W_lin2_attr, W_lin2_crystal)
    w_l3_flat = W_lin3.transpose(1, 0, 2).reshape(-1, D_IN)
    w2r = W_fc2.reshape(H, D_IN, D_EDGE).transpose(2, 0, 1).reshape(D_EDGE * H, D_IN)
    pad = EH_PAD - EH
    esrc = edge_src.astype(jnp.int32)
    edst = edge_dst.astype(jnp.int32)
    idx2 = []
    for h2 in range(2):
        sl = slice(h2 * EH, (h2 + 1) * EH)
        idx2.append((
            jnp.pad(esrc[sl], (0, pad)).reshape(EH_PAD // K, K),
            jnp.pad(edst[sl], (0, pad)).reshape(EH_PAD // K, K)))

    s, x0, x1 = _tc_node_front(node_input, node_attr, node_crystal_attr,
                               w_sc_cat, w_l1_cat)
    ys = []
    for h2 in range(2):
        w0, w1 = _tc_edge(edge_length_embedded, edge_attr, W_fc0, W_fc1,
                          w2r, h2)
        ys += list(_sc_message(x0, x1, w0, w1, *idx2[h2]))
    ys = [ys[0], ys[1], ys[2], ys[3]]
    return _tc_node_back(ys, node_attr, node_crystal_attr, sym_mask, s,
                         w_l2_cat, w_l3_flat)


# speedup vs baseline: 1.0098x; 1.0098x over previous
import functools
import math

import jax
import jax.numpy as jnp
from jax import lax
from jax.experimental import pallas as pl
from jax.experimental.pallas import tpu as pltpu
from jax.experimental.pallas import tpu_sc as plsc

N = 10000
E = 160000
D_IN = 256
D_ATTR = 8
D_EDGE = 4
N_BASIS = 10
H = 64
_SILU_NORM = 1.679177
_INV2048 = 1.0 / math.sqrt(256.0 * 8.0)
_W_SCALE = 1.0 / (8.0 * 2.0 * math.sqrt(32.0))
_C_S = math.sin(math.pi / 8.0)
_C_X = math.cos(math.pi / 8.0)

BN = 1000
BE = 640
CW = 128
K = 40
EH = E // 2
EH_PAD = 81920
ROWS_PER_TILE = 128
NCHUNK = 128
REAL_EDGE_BLOCKS = EH // BE


def _node_front_body(xin_ref, na_ref, nc_ref, wsc_ref, wl1_ref,
                     s_ref, x0_ref, x1_ref):
    xin = xin_ref[...]
    y = jnp.concatenate([na_ref[...], nc_ref[...]], axis=1)
    xcat = jnp.concatenate([xin * y[:, v:v + 1] for v in range(16)], axis=1)
    s = jnp.dot(xcat, wsc_ref[...], preferred_element_type=jnp.float32)
    x = jnp.dot(xcat, wl1_ref[...], preferred_element_type=jnp.float32)
    s_ref[...] = s * _INV2048
    x = x * _INV2048
    x0_ref[...] = x[:, :CW]
    x1_ref[...] = x[:, CW:]


def _edge_body(emb_ref, ea_ref, w0_ref, w1_ref, w2r_ref, o0_ref, o1_ref):
    emb = emb_ref[...]
    h = jnp.dot(emb, w0_ref[...], preferred_element_type=jnp.float32)
    h = h * (1.0 / math.sqrt(N_BASIS))
    h = jax.nn.silu(h) * _SILU_NORM
    h = jnp.dot(h, w1_ref[...], preferred_element_type=jnp.float32) * (1.0 / 8.0)
    h = jax.nn.silu(h) * _SILU_NORM
    ea = ea_ref[...]
    h = h * _W_SCALE
    g = jnp.concatenate([h * ea[:, v:v + 1] for v in range(D_EDGE)], axis=1)
    weff = jnp.dot(g, w2r_ref[...], preferred_element_type=jnp.float32)
    pad = pl.program_id(0) >= REAL_EDGE_BLOCKS
    weff = jnp.where(pad, 0.0, weff)
    o0_ref[...] = weff[:, :CW]
    o1_ref[...] = weff[:, CW:]


def _node_back_body(y0a_ref, y1a_ref, y0b_ref, y1b_ref, na_ref, nc_ref,
                    sym_ref, s_ref, wl2_ref, wl3_ref, out_ref):
    xm = jnp.concatenate([y0a_ref[...] + y0b_ref[...],
                          y1a_ref[...] + y1b_ref[...]], axis=1)
    a = jnp.concatenate([na_ref[...], nc_ref[...]], axis=1)
    xcat = jnp.concatenate([xm * a[:, v:v + 1] for v in range(16)], axis=1)
    x2 = jnp.dot(xcat, wl2_ref[...], preferred_element_type=jnp.float32)
    x2 = x2 * _INV2048
    sym = sym_ref[...]
    xcat2 = jnp.concatenate([x2 * sym[:, v:v + 1] for v in range(8)], axis=1)
    x3 = jnp.dot(xcat2, wl3_ref[...], preferred_element_type=jnp.float32)
    x3 = x3 * _INV2048
    out_ref[...] = _C_S * s_ref[...] + _C_X * x3


def _sc_message_body(x0_hbm, x1_hbm, w0_hbm, w1_hbm, src_hbm, dst_hbm,
                     y0_hbm, y1_hbm,
                     xg0, xg1, xg2, wv0, wv1, sb0, sb1,
                     idxs, idxd, idxd2, acc,
                     gs0, gs1, gs2, ws0, ws1, ss0, ss1):
    c = lax.axis_index("c")
    s = lax.axis_index("s")
    xgs = [xg0, xg1, xg2]
    wvs = [wv0, wv1]
    sbs = [sb0, sb1]
    gss = [gs0, gs1, gs2]
    wss = [ws0, ws1]
    sss = [ss0, ss1]
    GW = 16 * K

    def flat_off(j):
        return ((j // 16) % 2) * GW + (j % 16) * K

    def _sc_run(xt, wt, yt):
        t_base = s * NCHUNK * K

        def zb(r, carry):
            for g in range(8):
                sb0[r, pl.ds(g * 16, 16)] = jnp.zeros((16,), jnp.float32)
            return carry
        lax.fori_loop(0, K, zb, 0)
        for k in range(16):
            cid = s + 16 * k

            def zcopy(cid=cid):
                pltpu.sync_copy(sb0, acc.at[pl.ds(cid * K, K)])
            pl.when(cid < N // K)(zcopy)
        plsc.subcore_barrier()

        pltpu.sync_copy(src_hbm.at[pl.ds(t_base, GW)], idxs.at[pl.ds(0, GW)])
        pltpu.sync_copy(dst_hbm.at[pl.ds(t_base, GW)], idxd.at[pl.ds(0, GW)])
        for t in range(3):
            pltpu.async_copy(xt.at[idxs.at[pl.ds(t * K, K)]], xgs[t], gss[t])
        for t in range(2):
            pltpu.async_copy(wt.at[pl.ds((s * NCHUNK + t) * K, K)],
                             wvs[t], wss[t])

        def load_next_group(j):
            gp = j // 16 + 1
            half = (gp % 2) * GW
            pltpu.sync_copy(src_hbm.at[pl.ds(t_base + gp * GW, GW)],
                            idxs.at[pl.ds(half, GW)])
            pltpu.sync_copy(dst_hbm.at[pl.ds(t_base + gp * GW, GW)],
                            idxd.at[pl.ds(half, GW)])

        def process(j, t3, t2, in_loop):
            if in_loop:
                pl.when((j % 16 == 8) & (j < 112))(lambda: load_next_group(j))

            def drain_sc():
                pltpu.make_async_copy(sbs[t2], acc.at[idxd2.at[t2]],
                                      sss[t2]).wait()
            if isinstance(j, int):
                if j >= 2:
                    drain_sc()
            else:
                pl.when(j >= 2)(drain_sc)

            fo = flat_off(j)
            pltpu.make_async_copy(xt.at[idxs.at[pl.ds(fo, K)]],
                                  xgs[t3], gss[t3]).wait()
            pltpu.make_async_copy(wt.at[pl.ds((s * NCHUNK + j) * K, K)],
                                  wvs[t2], wss[t2]).wait()

            idxd2[t2, pl.ds(0, 16)] = idxd[pl.ds(fo, 16)]
            idxd2[t2, pl.ds(16, 16)] = idxd[pl.ds(fo + 16, 16)]
            idxd2[t2, pl.ds(K - 16, 16)] = idxd[pl.ds(fo + K - 16, 16)]

            def mp(r, carry):
                for g in range(8):
                    sl = pl.ds(g * 16, 16)
                    sbs[t2][r, sl] = xgs[t3][r, sl] * wvs[t2][r, sl]
                return carry
            lax.fori_loop(0, K, mp, 0)

            pltpu.async_copy(sbs[t2], acc.at[idxd2.at[t2]], sss[t2], add=True)
            if in_loop:
                def issue_next_gather():
                    pltpu.async_copy(xt.at[idxs.at[pl.ds(flat_off(j + 3), K)]],
                                     xgs[t3], gss[t3])
                pl.when(j + 3 < NCHUNK)(issue_next_gather)
                pltpu.async_copy(wt.at[pl.ds((s * NCHUNK + j + 2) * K, K)],
                                 wvs[t2], wss[t2])
            else:
                if j + 3 < NCHUNK:
                    pltpu.async_copy(xt.at[idxs.at[pl.ds(flat_off(j + 3), K)]],
                                     xgs[t3], gss[t3])
                if j + 2 < NCHUNK:
                    pltpu.async_copy(wt.at[pl.ds((s * NCHUNK + j + 2) * K, K)],
                                     wvs[t2], wss[t2])

        def step(si, carry):
            for u in range(6):
                process(si * 6 + u, u % 3, u % 2, True)
            return carry
        lax.fori_loop(0, (NCHUNK // 6), step, 0)
        for j in range((NCHUNK // 6) * 6, NCHUNK):
            process(j, j % 3, j % 2, False)
        for j in range(NCHUNK - 2, NCHUNK):
            t2 = j % 2
            pltpu.make_async_copy(sbs[t2], acc.at[idxd2.at[t2]],
                                  sss[t2]).wait()
        plsc.subcore_barrier()

        def cback(cid):
            pltpu.sync_copy(acc.at[pl.ds(cid * K, K)], sb0)
            pltpu.sync_copy(sb0, yt.at[pl.ds(cid * K, K)])
        for k in range(16):
            cid = s + 16 * k
            pl.when(cid < N // K)(lambda cid=cid: cback(cid))
        plsc.subcore_barrier()

    pl.when(c == 0)(lambda: _sc_run(x0_hbm, w0_hbm, y0_hbm))
    pl.when(c == 1)(lambda: _sc_run(x1_hbm, w1_hbm, y1_hbm))


def _tc_node_front(node_input, node_attr, node_crystal_attr, w_sc_cat, w_l1_cat):
    grid = N // BN
    return pl.pallas_call(
        _node_front_body,
        grid=(grid,),
        in_specs=[
            pl.BlockSpec((BN, D_IN), lambda i: (i, 0)),
            pl.BlockSpec((BN, D_ATTR), lambda i: (i, 0)),
            pl.BlockSpec((BN, D_ATTR), lambda i: (i, 0)),
            pl.BlockSpec((16 * D_IN, D_IN), lambda i: (0, 0)),
            pl.BlockSpec((16 * D_IN, D_IN), lambda i: (0, 0)),
        ],
        out_specs=[
            pl.BlockSpec((BN, D_IN), lambda i: (i, 0)),
            pl.BlockSpec((BN, CW), lambda i: (i, 0)),
            pl.BlockSpec((BN, CW), lambda i: (i, 0)),
        ],
        out_shape=[
            jax.ShapeDtypeStruct((N, D_IN), jnp.float32),
            jax.ShapeDtypeStruct((N, CW), jnp.float32),
            jax.ShapeDtypeStruct((N, CW), jnp.float32),
        ],
    )(node_input, node_attr, node_crystal_attr, w_sc_cat, w_l1_cat)


def _tc_edge(edge_length_embedded, edge_attr, w_fc0, w_fc1, w2r, h2):
    grid = EH_PAD // BE
    off = h2 * REAL_EDGE_BLOCKS
    return pl.pallas_call(
        _edge_body,
        grid=(grid,),
        in_specs=[
            pl.BlockSpec((BE, N_BASIS),
                         lambda i: (off + jnp.minimum(i, REAL_EDGE_BLOCKS - 1), 0)),
            pl.BlockSpec((BE, D_EDGE),
                         lambda i: (off + jnp.minimum(i, REAL_EDGE_BLOCKS - 1), 0)),
            pl.BlockSpec((N_BASIS, H), lambda i: (0, 0)),
            pl.BlockSpec((H, H), lambda i: (0, 0)),
            pl.BlockSpec((D_EDGE * H, D_IN), lambda i: (0, 0)),
        ],
        out_specs=[
            pl.BlockSpec((BE, CW), lambda i: (i, 0)),
            pl.BlockSpec((BE, CW), lambda i: (i, 0)),
        ],
        out_shape=[
            jax.ShapeDtypeStruct((EH_PAD, CW), jnp.float32),
            jax.ShapeDtypeStruct((EH_PAD, CW), jnp.float32),
        ],
    )(edge_length_embedded, edge_attr, w_fc0, w_fc1, w2r)


def _sc_message(x0, x1, w0, w1, src2, dst2):
    mesh = plsc.VectorSubcoreMesh(core_axis_name="c", subcore_axis_name="s")
    f = functools.partial(
        pl.kernel,
        out_type=[
            jax.ShapeDtypeStruct((N, CW), jnp.float32),
            jax.ShapeDtypeStruct((N, CW), jnp.float32),
        ],
        mesh=mesh,
        scratch_types=(
            [pltpu.VMEM((K, CW), jnp.float32) for _ in range(3)]
            + [pltpu.VMEM((K, CW), jnp.float32) for _ in range(2)]
            + [pltpu.VMEM((K, CW), jnp.float32) for _ in range(2)]
            + [pltpu.VMEM((2 * 16 * K,), jnp.int32) for _ in range(2)]
            + [pltpu.VMEM((2, K), jnp.int32)]
            + [pltpu.VMEM_SHARED((N, CW), jnp.float32)]
            + [pltpu.SemaphoreType.DMA for _ in range(7)]
        ),
    )(_sc_message_body)
    return f(x0, x1, w0, w1, src2, dst2)


def _tc_node_back(ys, node_attr, node_crystal_attr, sym_mask, s,
                  w_l2_cat, w_lin3):
    grid = N // BN
    return pl.pallas_call(
        _node_back_body,
        grid=(grid,),
        in_specs=[
            pl.BlockSpec((BN, CW), lambda i: (i, 0)),
            pl.BlockSpec((BN, CW), lambda i: (i, 0)),
            pl.BlockSpec((BN, CW), lambda i: (i, 0)),
            pl.BlockSpec((BN, CW), lambda i: (i, 0)),
            pl.BlockSpec((BN, D_ATTR), lambda i: (i, 0)),
            pl.BlockSpec((BN, D_ATTR), lambda i: (i, 0)),
            pl.BlockSpec((BN, D_ATTR), lambda i: (i, 0)),
            pl.BlockSpec((BN, D_IN), lambda i: (i, 0)),
            pl.BlockSpec((16 * D_IN, D_IN), lambda i: (0, 0)),
            pl.BlockSpec((D_ATTR * D_IN, D_IN), lambda i: (0, 0)),
        ],
        out_specs=pl.BlockSpec((BN, D_IN), lambda i: (i, 0)),
        out_shape=jax.ShapeDtypeStruct((N, D_IN), jnp.float32),
    )(*ys, node_attr, node_crystal_attr, sym_mask, s, w_l2_cat, w_lin3)


def kernel(node_input, node_attr, edge_attr, edge_length_embedded,
           node_crystal_attr, sym_mask, W_sc_attr, W_sc_crystal, W_lin1_attr,
           W_lin1_crystal, W_fc0, W_fc1, W_fc2, W_lin2_attr, W_lin2_crystal,
           W_lin3, edge_src, edge_dst):
    def vmajor(wa, wb):
        w = jnp.concatenate([wa, wb], axis=1)
        return w.transpose(1, 0, 2).reshape(-1, D_IN)

    w_sc_cat = vmajor(W_sc_attr, W_sc_crystal)
    w_l1_cat = vmajor(W_lin1_attr, W_lin1_crystal)
    w_l2_cat = vmajor(W_lin2_attr, W_lin2_crystal)
    w_l3_flat = W_lin3.transpose(1, 0, 2).reshape(-1, D_IN)
    w2r = W_fc2.reshape(H, D_IN, D_EDGE).transpose(2, 0, 1).reshape(D_EDGE * H, D_IN)
    pad = EH_PAD - EH
    esrc = edge_src.astype(jnp.int32)
    edst = edge_dst.astype(jnp.int32)
    idx2 = []
    for h2 in range(2):
        sl = slice(h2 * EH, (h2 + 1) * EH)
        idx2.append((jnp.pad(esrc[sl], (0, pad)),
                     jnp.pad(edst[sl], (0, pad))))

    s, x0, x1 = _tc_node_front(node_input, node_attr, node_crystal_attr,
                               w_sc_cat, w_l1_cat)
    ys = []
    for h2 in range(2):
        w0, w1 = _tc_edge(edge_length_embedded, edge_attr, W_fc0, W_fc1,
                          w2r, h2)
        ys += list(_sc_message(x0, x1, w0, w1, *idx2[h2]))
    ys = [ys[0], ys[1], ys[2], ys[3]]
    return _tc_node_back(ys, node_attr, node_crystal_attr, sym_mask, s,
                         w_l2_cat, w_l3_flat)


# speedup vs baseline: 1.0969x; 1.0863x over previous
import functools
import math

import jax
import jax.numpy as jnp
from jax import lax
from jax.experimental import pallas as pl
from jax.experimental.pallas import tpu as pltpu
from jax.experimental.pallas import tpu_sc as plsc

N = 10000
E = 160000
D_IN = 256
D_ATTR = 8
D_EDGE = 4
N_BASIS = 10
H = 64
_SILU_NORM = 1.679177
_INV2048 = 1.0 / math.sqrt(256.0 * 8.0)
_W_SCALE = 1.0 / (8.0 * 2.0 * math.sqrt(32.0))
_C_S = math.sin(math.pi / 8.0)
_C_X = math.cos(math.pi / 8.0)

BN = 1000
BE = 640
CW = 128
K = 40
EH = E // 2
EH_PAD = 81920
ROWS_PER_TILE = 128
NCHUNK = 128
REAL_EDGE_BLOCKS = EH // BE


def _node_front_body(xin_ref, na_ref, nc_ref, wsc_ref, wl1_ref,
                     s_ref, x0_ref, x1_ref):
    xin = xin_ref[...]
    y = jnp.concatenate([na_ref[...], nc_ref[...]], axis=1)
    xcat = jnp.concatenate([xin * y[:, v:v + 1] for v in range(16)], axis=1)
    s = jnp.dot(xcat, wsc_ref[...], preferred_element_type=jnp.float32)
    x = jnp.dot(xcat, wl1_ref[...], preferred_element_type=jnp.float32)
    s_ref[...] = s * _INV2048
    x = x * _INV2048
    x0_ref[...] = x[:, :CW]
    x1_ref[...] = x[:, CW:]


def _edge_body(emb_ref, ea_ref, w0_ref, w1_ref, w2r_ref, o0_ref, o1_ref):
    emb = jnp.transpose(emb_ref[...])
    h = jnp.dot(emb, w0_ref[...], preferred_element_type=jnp.float32)
    h = h * (1.0 / math.sqrt(N_BASIS))
    h = jax.nn.silu(h) * _SILU_NORM
    h = jnp.dot(h, w1_ref[...], preferred_element_type=jnp.float32) * (1.0 / 8.0)
    h = jax.nn.silu(h) * _SILU_NORM
    ea = jnp.transpose(ea_ref[...])
    h = h * _W_SCALE
    g = jnp.concatenate([h * ea[:, v:v + 1] for v in range(D_EDGE)], axis=1)
    weff = jnp.dot(g, w2r_ref[...], preferred_element_type=jnp.float32)
    pad = pl.program_id(0) >= REAL_EDGE_BLOCKS
    weff = jnp.where(pad, 0.0, weff)
    o0_ref[...] = weff[:, :CW]
    o1_ref[...] = weff[:, CW:]


def _node_back_body(y0a_ref, y1a_ref, y0b_ref, y1b_ref, na_ref, nc_ref,
                    sym_ref, s_ref, wl2_ref, wl3_ref, out_ref):
    xm = jnp.concatenate([y0a_ref[...] + y0b_ref[...],
                          y1a_ref[...] + y1b_ref[...]], axis=1)
    a = jnp.concatenate([na_ref[...], nc_ref[...]], axis=1)
    xcat = jnp.concatenate([xm * a[:, v:v + 1] for v in range(16)], axis=1)
    x2 = jnp.dot(xcat, wl2_ref[...], preferred_element_type=jnp.float32)
    x2 = x2 * _INV2048
    sym = sym_ref[...]
    xcat2 = jnp.concatenate([x2 * sym[:, v:v + 1] for v in range(8)], axis=1)
    x3 = jnp.dot(xcat2, wl3_ref[...], preferred_element_type=jnp.float32)
    x3 = x3 * _INV2048
    out_ref[...] = _C_S * s_ref[...] + _C_X * x3


def _sc_message_body(x0_hbm, x1_hbm, w0_hbm, w1_hbm, src_hbm, dst_hbm,
                     y0_hbm, y1_hbm,
                     xg0, xg1, xg2, wv0, wv1, sb0, sb1,
                     idxs, idxd, idxd2, acc,
                     gs0, gs1, gs2, ws0, ws1, ss0, ss1):
    c = lax.axis_index("c")
    s = lax.axis_index("s")
    xgs = [xg0, xg1, xg2]
    wvs = [wv0, wv1]
    sbs = [sb0, sb1]
    gss = [gs0, gs1, gs2]
    wss = [ws0, ws1]
    sss = [ss0, ss1]
    GW = 16 * K

    def flat_off(j):
        return ((j // 16) % 2) * GW + (j % 16) * K

    def _sc_run(xt, wt, yt):
        t_base = s * NCHUNK * K

        def zb(r, carry):
            for g in range(8):
                sb0[r, pl.ds(g * 16, 16)] = jnp.zeros((16,), jnp.float32)
            return carry
        lax.fori_loop(0, K, zb, 0)
        for k in range(16):
            cid = s + 16 * k

            def zcopy(cid=cid):
                pltpu.sync_copy(sb0, acc.at[pl.ds(cid * K, K)])
            pl.when(cid < N // K)(zcopy)
        plsc.subcore_barrier()

        pltpu.sync_copy(src_hbm.at[pl.ds(t_base, GW)], idxs.at[pl.ds(0, GW)])
        pltpu.sync_copy(dst_hbm.at[pl.ds(t_base, GW)], idxd.at[pl.ds(0, GW)])
        for t in range(3):
            pltpu.async_copy(xt.at[idxs.at[pl.ds(t * K, K)]], xgs[t], gss[t])
        for t in range(2):
            pltpu.async_copy(wt.at[pl.ds((s * NCHUNK + t) * K, K)],
                             wvs[t], wss[t])

        def load_next_group(j):
            gp = j // 16 + 1
            half = (gp % 2) * GW
            pltpu.sync_copy(src_hbm.at[pl.ds(t_base + gp * GW, GW)],
                            idxs.at[pl.ds(half, GW)])
            pltpu.sync_copy(dst_hbm.at[pl.ds(t_base + gp * GW, GW)],
                            idxd.at[pl.ds(half, GW)])

        def process(j, t3, t2, in_loop):
            if in_loop:
                pl.when((j % 16 == 8) & (j < 112))(lambda: load_next_group(j))

            def drain_sc():
                pltpu.make_async_copy(sbs[t2], acc.at[idxd2.at[t2]],
                                      sss[t2]).wait()
            if isinstance(j, int):
                if j >= 2:
                    drain_sc()
            else:
                pl.when(j >= 2)(drain_sc)

            fo = flat_off(j)
            pltpu.make_async_copy(xt.at[idxs.at[pl.ds(fo, K)]],
                                  xgs[t3], gss[t3]).wait()
            pltpu.make_async_copy(wt.at[pl.ds((s * NCHUNK + j) * K, K)],
                                  wvs[t2], wss[t2]).wait()

            idxd2[t2, pl.ds(0, 16)] = idxd[pl.ds(fo, 16)]
            idxd2[t2, pl.ds(16, 16)] = idxd[pl.ds(fo + 16, 16)]
            idxd2[t2, pl.ds(K - 16, 16)] = idxd[pl.ds(fo + K - 16, 16)]

            def mp(r, carry):
                for g in range(8):
                    sl = pl.ds(g * 16, 16)
                    sbs[t2][r, sl] = xgs[t3][r, sl] * wvs[t2][r, sl]
                return carry
            lax.fori_loop(0, K, mp, 0)

            pltpu.async_copy(sbs[t2], acc.at[idxd2.at[t2]], sss[t2], add=True)
            if in_loop:
                def issue_next_gather():
                    pltpu.async_copy(xt.at[idxs.at[pl.ds(flat_off(j + 3), K)]],
                                     xgs[t3], gss[t3])
                pl.when(j + 3 < NCHUNK)(issue_next_gather)
                pltpu.async_copy(wt.at[pl.ds((s * NCHUNK + j + 2) * K, K)],
                                 wvs[t2], wss[t2])
            else:
                if j + 3 < NCHUNK:
                    pltpu.async_copy(xt.at[idxs.at[pl.ds(flat_off(j + 3), K)]],
                                     xgs[t3], gss[t3])
                if j + 2 < NCHUNK:
                    pltpu.async_copy(wt.at[pl.ds((s * NCHUNK + j + 2) * K, K)],
                                     wvs[t2], wss[t2])

        def step(si, carry):
            for u in range(6):
                process(si * 6 + u, u % 3, u % 2, True)
            return carry
        lax.fori_loop(0, (NCHUNK // 6), step, 0)
        for j in range((NCHUNK // 6) * 6, NCHUNK):
            process(j, j % 3, j % 2, False)
        for j in range(NCHUNK - 2, NCHUNK):
            t2 = j % 2
            pltpu.make_async_copy(sbs[t2], acc.at[idxd2.at[t2]],
                                  sss[t2]).wait()
        plsc.subcore_barrier()

        def cback(cid):
            pltpu.sync_copy(acc.at[pl.ds(cid * K, K)], sb0)
            pltpu.sync_copy(sb0, yt.at[pl.ds(cid * K, K)])
        for k in range(16):
            cid = s + 16 * k
            pl.when(cid < N // K)(lambda cid=cid: cback(cid))
        plsc.subcore_barrier()

    pl.when(c == 0)(lambda: _sc_run(x0_hbm, w0_hbm, y0_hbm))
    pl.when(c == 1)(lambda: _sc_run(x1_hbm, w1_hbm, y1_hbm))


def _tc_node_front(node_input, node_attr, node_crystal_attr, w_sc_cat, w_l1_cat):
    grid = N // BN
    return pl.pallas_call(
        _node_front_body,
        grid=(grid,),
        in_specs=[
            pl.BlockSpec((BN, D_IN), lambda i: (i, 0)),
            pl.BlockSpec((BN, D_ATTR), lambda i: (i, 0)),
            pl.BlockSpec((BN, D_ATTR), lambda i: (i, 0)),
            pl.BlockSpec((16 * D_IN, D_IN), lambda i: (0, 0)),
            pl.BlockSpec((16 * D_IN, D_IN), lambda i: (0, 0)),
        ],
        out_specs=[
            pl.BlockSpec((BN, D_IN), lambda i: (i, 0)),
            pl.BlockSpec((BN, CW), lambda i: (i, 0)),
            pl.BlockSpec((BN, CW), lambda i: (i, 0)),
        ],
        out_shape=[
            jax.ShapeDtypeStruct((N, D_IN), jnp.float32),
            jax.ShapeDtypeStruct((N, CW), jnp.float32),
            jax.ShapeDtypeStruct((N, CW), jnp.float32),
        ],
    )(node_input, node_attr, node_crystal_attr, w_sc_cat, w_l1_cat)


def _tc_edge(edge_length_embedded, edge_attr, w_fc0, w_fc1, w2r, h2):
    grid = EH_PAD // BE
    off = h2 * REAL_EDGE_BLOCKS
    return pl.pallas_call(
        _edge_body,
        grid=(grid,),
        in_specs=[
            pl.BlockSpec((N_BASIS, BE),
                         lambda i: (0, off + jnp.minimum(i, REAL_EDGE_BLOCKS - 1))),
            pl.BlockSpec((D_EDGE, BE),
                         lambda i: (0, off + jnp.minimum(i, REAL_EDGE_BLOCKS - 1))),
            pl.BlockSpec((N_BASIS, H), lambda i: (0, 0)),
            pl.BlockSpec((H, H), lambda i: (0, 0)),
            pl.BlockSpec((D_EDGE * H, D_IN), lambda i: (0, 0)),
        ],
        out_specs=[
            pl.BlockSpec((BE, CW), lambda i: (i, 0)),
            pl.BlockSpec((BE, CW), lambda i: (i, 0)),
        ],
        out_shape=[
            jax.ShapeDtypeStruct((EH_PAD, CW), jnp.float32),
            jax.ShapeDtypeStruct((EH_PAD, CW), jnp.float32),
        ],
    )(edge_length_embedded, edge_attr, w_fc0, w_fc1, w2r)


def _sc_message(x0, x1, w0, w1, src2, dst2):
    mesh = plsc.VectorSubcoreMesh(core_axis_name="c", subcore_axis_name="s")
    f = functools.partial(
        pl.kernel,
        out_type=[
            jax.ShapeDtypeStruct((N, CW), jnp.float32),
            jax.ShapeDtypeStruct((N, CW), jnp.float32),
        ],
        mesh=mesh,
        scratch_types=(
            [pltpu.VMEM((K, CW), jnp.float32) for _ in range(3)]
            + [pltpu.VMEM((K, CW), jnp.float32) for _ in range(2)]
            + [pltpu.VMEM((K, CW), jnp.float32) for _ in range(2)]
            + [pltpu.VMEM((2 * 16 * K,), jnp.int32) for _ in range(2)]
            + [pltpu.VMEM((2, K), jnp.int32)]
            + [pltpu.VMEM_SHARED((N, CW), jnp.float32)]
            + [pltpu.SemaphoreType.DMA for _ in range(7)]
        ),
    )(_sc_message_body)
    return f(x0, x1, w0, w1, src2, dst2)


def _tc_node_back(ys, node_attr, node_crystal_attr, sym_mask, s,
                  w_l2_cat, w_lin3):
    grid = N // BN
    return pl.pallas_call(
        _node_back_body,
        grid=(grid,),
        in_specs=[
            pl.BlockSpec((BN, CW), lambda i: (i, 0)),
            pl.BlockSpec((BN, CW), lambda i: (i, 0)),
            pl.BlockSpec((BN, CW), lambda i: (i, 0)),
            pl.BlockSpec((BN, CW), lambda i: (i, 0)),
            pl.BlockSpec((BN, D_ATTR), lambda i: (i, 0)),
            pl.BlockSpec((BN, D_ATTR), lambda i: (i, 0)),
            pl.BlockSpec((BN, D_ATTR), lambda i: (i, 0)),
            pl.BlockSpec((BN, D_IN), lambda i: (i, 0)),
            pl.BlockSpec((16 * D_IN, D_IN), lambda i: (0, 0)),
            pl.BlockSpec((D_ATTR * D_IN, D_IN), lambda i: (0, 0)),
        ],
        out_specs=pl.BlockSpec((BN, D_IN), lambda i: (i, 0)),
        out_shape=jax.ShapeDtypeStruct((N, D_IN), jnp.float32),
    )(*ys, node_attr, node_crystal_attr, sym_mask, s, w_l2_cat, w_lin3)


def kernel(node_input, node_attr, edge_attr, edge_length_embedded,
           node_crystal_attr, sym_mask, W_sc_attr, W_sc_crystal, W_lin1_attr,
           W_lin1_crystal, W_fc0, W_fc1, W_fc2, W_lin2_attr, W_lin2_crystal,
           W_lin3, edge_src, edge_dst):
    def vmajor(wa, wb):
        w = jnp.concatenate([wa, wb], axis=1)
        return w.transpose(1, 0, 2).reshape(-1, D_IN)

    w_sc_cat = vmajor(W_sc_attr, W_sc_crystal)
    w_l1_cat = vmajor(W_lin1_attr, W_lin1_crystal)
    w_l2_cat = vmajor(W_lin2_attr, W_lin2_crystal)
    w_l3_flat = W_lin3.transpose(1, 0, 2).reshape(-1, D_IN)
    w2r = W_fc2.reshape(H, D_IN, D_EDGE).transpose(2, 0, 1).reshape(D_EDGE * H, D_IN)
    pad = EH_PAD - EH
    esrc = edge_src.astype(jnp.int32)
    edst = edge_dst.astype(jnp.int32)
    idx2 = []
    for h2 in range(2):
        sl = slice(h2 * EH, (h2 + 1) * EH)
        idx2.append((jnp.pad(esrc[sl], (0, pad)),
                     jnp.pad(edst[sl], (0, pad))))

    embT = edge_length_embedded.T
    eaT = edge_attr.T
    s, x0, x1 = _tc_node_front(node_input, node_attr, node_crystal_attr,
                               w_sc_cat, w_l1_cat)
    ys = []
    for h2 in range(2):
        w0, w1 = _tc_edge(embT, eaT, W_fc0, W_fc1, w2r, h2)
        ys += list(_sc_message(x0, x1, w0, w1, *idx2[h2]))
    ys = [ys[0], ys[1], ys[2], ys[3]]
    return _tc_node_back(ys, node_attr, node_crystal_attr, sym_mask, s,
                         w_l2_cat, w_l3_flat)
